# Initial kernel scaffold; baseline (speedup 1.0000x reference)
import jax
import jax.numpy as jnp
from jax.experimental import pallas as pl


def kernel(nodes, edges, senders, receivers, enc_W, enc_b, core0_W, core0_b, core1_W, core1_b, dec_W, dec_b):
    raise NotImplementedError("write your pallas kernel here")


# speedup vs baseline: 3.7995x; 3.7995x over previous
import functools

import jax
import jax.numpy as jnp
from jax import lax
from jax.experimental import pallas as pl
from jax.experimental.pallas import tpu as pltpu
from jax.experimental.pallas import tpu_sc as plsc

N = 10000
E = 320000
D = 128
L = 128
C = 40

NC = 2
NS = 16
NW = NC * NS
EPW = E // NW
K = 80
NCHUNK = EPW // K
NSIO = 10
RPS = N // NSIO
ZROWS = 200


def _hop_body(x_hbm, w_hbm, recv_hbm, send_hbm, out_hbm,
              ridx_v, sidx_v, w_v, rows_v, zb_v, acc_sh, sem):
    cid = lax.axis_index("c")
    sid = lax.axis_index("s")
    wid = cid * NS + sid

    zvec = jnp.zeros((16,), jnp.float32)

    @pl.when(sid < NSIO)
    def _zero():
        @pl.loop(0, ZROWS * (D // 16))
        def _zero_fill(i):
            r = i // (D // 16)
            c = i % (D // 16)
            zb_v[r, pl.ds(pl.multiple_of(c * 16, 16), 16)] = zvec

        @pl.loop(0, RPS // ZROWS)
        def _zero_acc(j):
            pltpu.sync_copy(zb_v,
                            acc_sh.at[pl.ds(sid * RPS + j * ZROWS, ZROWS)])

    plsc.subcore_barrier()

    base0 = wid * EPW

    @pl.loop(0, NCHUNK)
    def _chunk(i):
        base = base0 + i * K
        pltpu.sync_copy(recv_hbm.at[pl.ds(base, K)], ridx_v)
        pltpu.sync_copy(send_hbm.at[pl.ds(base, K)], sidx_v)
        pltpu.sync_copy(w_hbm.at[pl.ds(base, K)], w_v)
        pltpu.async_copy(x_hbm.at[ridx_v], rows_v, sem).wait()

        @pl.loop(0, K // 16)
        def _scale(g):
            w16 = w_v[pl.ds(pl.multiple_of(g * 16, 16), 16)]
            for t in range(16):
                wk = w16[t]
                for j in range(D // 16):
                    sl = pl.ds(j * 16, 16)
                    rows_v[g * 16 + t, sl] = rows_v[g * 16 + t, sl] * wk

        pltpu.sync_copy(rows_v, acc_sh.at[sidx_v], add=True)

    plsc.subcore_barrier()

    @pl.when(sid < NSIO)
    def _copy_out():
        pltpu.sync_copy(acc_sh.at[pl.ds(sid * RPS, RPS)],
                        out_hbm.at[cid, pl.ds(sid * RPS, RPS)])


_hop = functools.partial(
    pl.kernel,
    out_type=jax.ShapeDtypeStruct((NC, N, L), jnp.float32),
    mesh=plsc.VectorSubcoreMesh(core_axis_name="c", subcore_axis_name="s",
                                num_cores=NC, num_subcores=NS),
    scratch_types=[
        pltpu.VMEM((K,), jnp.int32),
        pltpu.VMEM((K,), jnp.int32),
        pltpu.VMEM((K,), jnp.float32),
        pltpu.VMEM((K, L), jnp.float32),
        pltpu.VMEM((ZROWS, L), jnp.float32),
        pltpu.VMEM_SHARED((N, L), jnp.float32),
        pltpu.SemaphoreType.DMA,
    ],
)(_hop_body)


BM = 2000


def _encoder_body(x_ref, w_ref, b_ref, o_ref):
    y = jnp.dot(x_ref[...], w_ref[...], preferred_element_type=jnp.float32)
    o_ref[...] = jnp.maximum(y + b_ref[...], 0.0)


def _encoder(x, w, b):
    return pl.pallas_call(
        _encoder_body,
        grid=(N // BM,),
        in_specs=[
            pl.BlockSpec((BM, D), lambda i: (i, 0)),
            pl.BlockSpec((D, L), lambda i: (0, 0)),
            pl.BlockSpec((1, L), lambda i: (0, 0)),
        ],
        out_specs=pl.BlockSpec((BM, L), lambda i: (i, 0)),
        out_shape=jax.ShapeDtypeStruct((N, L), jnp.float32),
    )(x, w, b.reshape(1, L))


def _update_body(p_ref, w_ref, b_ref, o_ref):
    conv = p_ref[0] + p_ref[1]
    h = jnp.dot(conv, w_ref[...], preferred_element_type=jnp.float32)
    o_ref[...] = jnp.maximum(h + b_ref[...], 0.0) + conv


def _update(parts, w, b):
    return pl.pallas_call(
        _update_body,
        grid=(N // BM,),
        in_specs=[
            pl.BlockSpec((NC, BM, L), lambda i: (0, i, 0)),
            pl.BlockSpec((L, L), lambda i: (0, 0)),
            pl.BlockSpec((1, L), lambda i: (0, 0)),
        ],
        out_specs=pl.BlockSpec((BM, L), lambda i: (i, 0)),
        out_shape=jax.ShapeDtypeStruct((N, L), jnp.float32),
    )(parts, w, b.reshape(1, L))


def _update_dec_body(p_ref, w_ref, b_ref, dw_ref, db_ref, o_ref):
    conv = p_ref[0] + p_ref[1]
    h = jnp.dot(conv, w_ref[...], preferred_element_type=jnp.float32)
    x = jnp.maximum(h + b_ref[...], 0.0) + conv
    o_ref[...] = jnp.dot(x, dw_ref[...],
                         preferred_element_type=jnp.float32) + db_ref[...]


def _update_dec(parts, w, b, dw, db):
    return pl.pallas_call(
        _update_dec_body,
        grid=(N // BM,),
        in_specs=[
            pl.BlockSpec((NC, BM, L), lambda i: (0, i, 0)),
            pl.BlockSpec((L, L), lambda i: (0, 0)),
            pl.BlockSpec((1, L), lambda i: (0, 0)),
            pl.BlockSpec((L, L), lambda i: (0, 0)),
            pl.BlockSpec((1, L), lambda i: (0, 0)),
        ],
        out_specs=pl.BlockSpec((BM, L), lambda i: (i, 0)),
        out_shape=jax.ShapeDtypeStruct((N, L), jnp.float32),
    )(parts, w, b.reshape(1, L), dw, db.reshape(1, L))


def kernel(nodes, edges, senders, receivers, enc_W, enc_b, core0_W, core0_b,
           core1_W, core1_b, dec_W, dec_b):
    w = edges.reshape(E)
    senders = senders.astype(jnp.int32)
    receivers = receivers.astype(jnp.int32)

    x = _encoder(nodes, enc_W, enc_b)
    parts = _hop(x, w, receivers, senders)
    x = _update(parts, core0_W, core0_b)
    parts = _hop(x, w, receivers, senders)

    dw_pad = jnp.zeros((L, L), jnp.float32).at[:, :C].set(dec_W)
    db_pad = jnp.zeros((L,), jnp.float32).at[:C].set(dec_b)
    out = _update_dec(parts, core1_W, core1_b, dw_pad, db_pad)
    return out[:, :C]
